# Initial kernel scaffold; baseline (speedup 1.0000x reference)
import jax
import jax.numpy as jnp
from jax.experimental import pallas as pl


def kernel(data, t0, tn, beta, z0):
    raise NotImplementedError("write your pallas kernel here")


# speedup vs baseline: 5.5838x; 5.5838x over previous
import functools

import jax
import jax.numpy as jnp
from jax import lax
from jax.experimental import pallas as pl
from jax.experimental.pallas import tpu as pltpu
from jax.experimental.pallas import tpu_sc as plsc

_L = 16
_NC = 2
_NS = 16
_NW = _NC * _NS


@functools.cache
def _build(n_events: int, n_nodes: int):
    assert n_events % (_NW * _L) == 0
    ev_per_w = n_events // _NW
    n_iter = ev_per_w // _L
    mesh = plsc.VectorSubcoreMesh(core_axis_name="c", subcore_axis_name="s")

    @functools.partial(
        pl.kernel,
        out_type=[
            jax.ShapeDtypeStruct((_NW * _L,), jnp.float32),
            jax.ShapeDtypeStruct((_NW * _L,), jnp.float32),
        ],
        mesh=mesh,
        scratch_types=[
            pltpu.VMEM((ev_per_w,), jnp.int32),
            pltpu.VMEM((ev_per_w,), jnp.int32),
            pltpu.VMEM((n_nodes,), jnp.float32),
            pltpu.VMEM((n_nodes,), jnp.float32),
            pltpu.VMEM((_L,), jnp.float32),
            pltpu.VMEM((_L,), jnp.float32),
        ],
        compiler_params=pltpu.CompilerParams(needs_layout_passes=False),
    )
    def sc_kernel(i_hbm, j_hbm, x_hbm, y_hbm, b_hbm, ev_out, ne_out,
                  i_v, j_v, x_v, y_v, oa_v, ob_v):
        wid = lax.axis_index("s") * _NC + lax.axis_index("c")
        base = wid * ev_per_w
        pltpu.sync_copy(x_hbm, x_v)
        pltpu.sync_copy(y_hbm, y_v)
        pltpu.sync_copy(b_hbm, oa_v)
        pltpu.sync_copy(i_hbm.at[pl.ds(base, ev_per_w)], i_v)
        pltpu.sync_copy(j_hbm.at[pl.ds(base, ev_per_w)], j_v)
        bvec = oa_v[...]

        def body(t, carry):
            acc_ev, acc_ne = carry
            iv = i_v[pl.ds(t * _L, _L)]
            jv = j_v[pl.ds(t * _L, _L)]
            xi = plsc.load_gather(x_v, [iv])
            yi = plsc.load_gather(y_v, [iv])
            xj = plsc.load_gather(x_v, [jv])
            yj = plsc.load_gather(y_v, [jv])
            dx = xi - xj
            dy = yi - yj
            lam = bvec - (dx * dx + dy * dy)
            return acc_ev + lam, acc_ne + jnp.exp(lam)

        zero = jnp.zeros((_L,), jnp.float32)
        acc_ev, acc_ne = lax.fori_loop(0, n_iter, body, (zero, zero))
        oa_v[...] = acc_ev
        ob_v[...] = acc_ne
        pltpu.sync_copy(oa_v, ev_out.at[pl.ds(wid * _L, _L)])
        pltpu.sync_copy(ob_v, ne_out.at[pl.ds(wid * _L, _L)])

    return sc_kernel


def kernel(data, t0, tn, beta, z0):
    n_events = data.shape[0]
    n_nodes = z0.shape[0]
    i_arr = data[:, 0].astype(jnp.int32)
    j_arr = data[:, 1].astype(jnp.int32)
    x_arr = z0[:, 0].astype(jnp.float32)
    y_arr = z0[:, 1].astype(jnp.float32)
    b_vec = jnp.broadcast_to(jnp.reshape(beta, (-1,))[0].astype(jnp.float32),
                             (_L,))
    ev_part, ne_part = _build(n_events, n_nodes)(i_arr, j_arr, x_arr, y_arr,
                                                 b_vec)
    return -(jnp.sum(ev_part) - jnp.sum(ne_part))
